# baseline (speedup 1.0000x reference)
import functools

import jax
import jax.numpy as jnp
from jax import lax
from jax.experimental import pallas as pl
from jax.experimental.pallas import tpu as pltpu
from jax.experimental.pallas import tpu_sc as plsc

N_NODES = 10000
N_EDGES = 160000
NUM_REL = 2
NUM_GRAPHS = 64
HIDDEN = 512
DIN = 256
DINA = DIN + 128
NP = 10240
RB = 320
NSUB = 32
EPAD = 160768
C1 = 2000
OBUF = 3328
NPR = NP * NUM_REL


def _wid():
    return lax.axis_index("s") * 2 + lax.axis_index("c")


def _sc_bucket(src, dst, etype):
    mesh = plsc.VectorSubcoreMesh(core_axis_name="c", subcore_axis_name="s")

    @functools.partial(
        pl.kernel,
        out_type=(
            jax.ShapeDtypeStruct((NSUB * EPAD,), jnp.int32),
            jax.ShapeDtypeStruct((NSUB * EPAD,), jnp.int32),
            jax.ShapeDtypeStruct((NSUB * 16,), jnp.int32),
        ),
        mesh=mesh,
        compiler_params=pltpu.CompilerParams(needs_layout_passes=False),
        scratch_types=[
            pltpu.VMEM((C1,), jnp.int32),
            pltpu.VMEM((C1,), jnp.int32),
            pltpu.VMEM((C1,), jnp.int32),
            pltpu.VMEM((OBUF,), jnp.int32),
            pltpu.VMEM((OBUF,), jnp.int32),
            pltpu.VMEM((16,), jnp.int32),
        ],
    )
    def k(src_hbm, dst_hbm, typ_hbm, srcc_hbm, slotc_hbm, cnt_hbm,
          sbuf, dbuf, tbuf, osrc, oslot, cvec):
        w = _wid()
        base = w * RB

        def chunk_body(i, carry):
            ocnt, flushed = carry
            pltpu.sync_copy(src_hbm.at[pl.ds(i * C1, C1)], sbuf)
            pltpu.sync_copy(dst_hbm.at[pl.ds(i * C1, C1)], dbuf)
            pltpu.sync_copy(typ_hbm.at[pl.ds(i * C1, C1)], tbuf)

            def grp(g, ocnt):
                off = g * 16
                d = dbuf[pl.ds(off, 16)]
                s = sbuf[pl.ds(off, 16)]
                t = tbuf[pl.ds(off, 16)]
                m = (d >= base) & (d < base + RB)
                row = d * NUM_REL + t
                cum = plsc.cumsum(jnp.where(m, 1, 0))
                idx = cum + (ocnt - 1)
                plsc.store_scatter(osrc, [idx], s, mask=m)
                plsc.store_scatter(oslot, [idx], row, mask=m)
                return ocnt + jnp.max(plsc.all_reduce_population_count(m))

            ocnt = lax.fori_loop(0, C1 // 16, grp, ocnt)

            for _rep in range(2):
                do_flush = ocnt >= 1024

                @pl.when(do_flush)
                def _():
                    o = pl.multiple_of(w * EPAD + flushed, 1024)
                    pltpu.sync_copy(osrc.at[pl.ds(0, 1024)],
                                    srcc_hbm.at[pl.ds(o, 1024)])
                    pltpu.sync_copy(oslot.at[pl.ds(0, 1024)],
                                    slotc_hbm.at[pl.ds(o, 1024)])

                    def mv(j, _):
                        osrc[pl.ds(j * 16, 16)] = osrc[pl.ds(1024 + j * 16, 16)]
                        oslot[pl.ds(j * 16, 16)] = oslot[pl.ds(1024 + j * 16, 16)]
                        return 0

                    lax.fori_loop(0, 128, mv, 0)

                ocnt = jnp.where(do_flush, ocnt - 1024, ocnt)
                flushed = jnp.where(do_flush, flushed + 1024, flushed)
            return ocnt, flushed

        ocnt, flushed = lax.fori_loop(0, N_EDGES // C1, chunk_body,
                                      (jnp.int32(0), jnp.int32(0)))

        zeros = jnp.zeros((16,), jnp.int32)
        trash = jnp.zeros((16,), jnp.int32) + (N_NODES * NUM_REL + w)
        lane = lax.iota(jnp.int32, 16)
        for kk in range(17):
            idx = lane + (ocnt + kk * 16)
            plsc.store_scatter(osrc, [idx], zeros)
            plsc.store_scatter(oslot, [idx], trash)
        padded = ((lax.max(ocnt, 1) + 255) // 256) * 256

        def fl(j, _):
            o = pl.multiple_of(w * EPAD + flushed + j * 128, 128)
            pltpu.sync_copy(osrc.at[pl.ds(j * 128, 128)],
                            srcc_hbm.at[pl.ds(o, 128)])
            pltpu.sync_copy(oslot.at[pl.ds(j * 128, 128)],
                            slotc_hbm.at[pl.ds(o, 128)])
            return 0

        lax.fori_loop(0, padded // 128, fl, 0)
        cvec[...] = jnp.full((16,), flushed + padded, jnp.int32)
        pltpu.sync_copy(cvec, cnt_hbm.at[pl.ds(pl.multiple_of(w * 16, 16), 16)])

    return k(src, dst, etype)


def _sc_aggregate(h, srcc, rowc, counts, din):
    KC = 128 if din <= 256 else 64
    mesh = plsc.VectorSubcoreMesh(core_axis_name="c", subcore_axis_name="s")
    out_type = [jax.ShapeDtypeStruct((NPR, din), jnp.float32)]
    scratch = [
        pltpu.VMEM((KC, din), jnp.float32),
        pltpu.VMEM((KC, din), jnp.float32),
        pltpu.VMEM((KC,), jnp.int32),
        pltpu.VMEM((KC,), jnp.int32),
        pltpu.VMEM((KC,), jnp.int32),
        pltpu.VMEM((KC,), jnp.int32),
        pltpu.VMEM((16,), jnp.int32),
        pltpu.SemaphoreType.DMA,
        pltpu.SemaphoreType.DMA,
        pltpu.SemaphoreType.DMA,
        pltpu.SemaphoreType.DMA,
    ]

    def body(h_hbm, srcc_hbm, rowc_hbm, cnt_hbm, agg_hbm, *rest):
        (rows0, rows1, sbuf0, sbuf1, rbuf0, rbuf1, cntv,
         gsem0, gsem1, ssem0, ssem1) = rest[:11]
        rows = (rows0, rows1)
        sbufs = (sbuf0, sbuf1)
        rbufs = (rbuf0, rbuf1)
        gsems = (gsem0, gsem1)
        ssems = (ssem0, ssem1)
        w = _wid()
        base = w * RB * NUM_REL
        pltpu.sync_copy(cnt_hbm.at[pl.ds(pl.multiple_of(w * 16, 16), 16)], cntv)
        total = jnp.max(cntv[...])
        npairs = total // (2 * KC)

        zf = jnp.zeros((16,), jnp.float32)

        def zb(i, _):
            for g in range(din // 16):
                rows0[i, pl.ds(g * 16, 16)] = zf
            return 0

        lax.fori_loop(0, KC, zb, 0)
        for j in range(RB * NUM_REL // KC):
            pltpu.sync_copy(
                rows0,
                agg_hbm.at[pl.ds(pl.multiple_of(base + j * KC, 8), KC)])

        def load_idx(c, s):
            o = pl.multiple_of(w * EPAD + c * KC, KC)
            pltpu.sync_copy(srcc_hbm.at[pl.ds(o, KC)], sbufs[s])
            pltpu.sync_copy(rowc_hbm.at[pl.ds(o, KC)], rbufs[s])

        def start_gather(s):
            pltpu.async_copy(h_hbm.at[sbufs[s]], rows[s], gsems[s])

        def wait_gather(s):
            pltpu.make_async_copy(h_hbm.at[sbufs[s]], rows[s], gsems[s]).wait()

        def start_scatter(s):
            pltpu.async_copy(rows[s], agg_hbm.at[rbufs[s]], ssems[s], add=True)

        def wait_scatter(s):
            pltpu.make_async_copy(rows[s], agg_hbm.at[rbufs[s]],
                                  ssems[s]).wait()

        load_idx(0, 0)
        start_gather(0)
        load_idx(1, 1)
        start_gather(1)

        def pb(p, _):
            wait_gather(0)
            start_scatter(0)
            wait_gather(1)
            wait_scatter(0)
            start_scatter(1)

            @pl.when(p < npairs - 1)
            def _():
                load_idx(2 * p + 2, 0)
                start_gather(0)

            wait_scatter(1)

            @pl.when(p < npairs - 1)
            def _():
                load_idx(2 * p + 3, 1)
                start_gather(1)

            return 0

        lax.fori_loop(0, npairs, pb, 0)

    f = pl.kernel(body, out_type=tuple(out_type), mesh=mesh,
                  scratch_types=scratch,
                  compiler_params=pltpu.CompilerParams(
                      needs_layout_passes=False))
    return f(h, srcc, rowc, counts)[0]


def _elu(x):
    return jnp.where(x > 0, x, jnp.exp(jnp.minimum(x, 0.0)) - 1.0)


def _rgcn_block(h_ref, agg_ref, cnt_ref, root_ref, W_ref, b_ref):
    din = root_ref.shape[0]
    out = jnp.dot(h_ref[...], root_ref[...], preferred_element_type=jnp.float32)
    for r in range(NUM_REL):
        icr = 1.0 / jnp.maximum(cnt_ref[:, r, 0], 1.0)
        a = agg_ref[:, r, :din] * icr[:, None]
        out = out + jnp.dot(a, W_ref[r], preferred_element_type=jnp.float32)
    return _elu(out + b_ref[...])


def _mid_layer_body(h_ref, agg_ref, cnt_ref, root_ref, W_ref, b_ref, out_ref):
    out_ref[...] = _rgcn_block(h_ref, agg_ref, cnt_ref, root_ref, W_ref, b_ref)


def _tc_mid_layer(h, agg, cnt, root, W, b):
    din = h.shape[1]
    wa = agg.shape[2]
    return pl.pallas_call(
        _mid_layer_body,
        grid=(NP // RB,),
        in_specs=[
            pl.BlockSpec((RB, din), lambda i: (i, 0)),
            pl.BlockSpec((RB, NUM_REL, wa), lambda i: (i, 0, 0)),
            pl.BlockSpec((RB, NUM_REL, 128), lambda i: (i, 0, DIN // 128)),
            pl.BlockSpec((din, HIDDEN), lambda i: (0, 0)),
            pl.BlockSpec((NUM_REL, din, HIDDEN), lambda i: (0, 0, 0)),
            pl.BlockSpec((1, HIDDEN), lambda i: (0, 0)),
        ],
        out_specs=pl.BlockSpec((RB, HIDDEN), lambda i: (i, 0)),
        out_shape=jax.ShapeDtypeStruct((NP, HIDDEN), jnp.float32),
    )(h, agg, cnt, root, W, b)


def _last_layer_body(h_ref, agg_ref, cnt_ref, root_ref, W_ref, b_ref,
                     oh_ref, lw_ref, lb_ref, out_ref, sum_scr, cnt_scr):
    i = pl.program_id(0)

    @pl.when(i == 0)
    def _():
        sum_scr[...] = jnp.zeros_like(sum_scr)
        cnt_scr[...] = jnp.zeros_like(cnt_scr)

    hout = _rgcn_block(h_ref, agg_ref, cnt_ref, root_ref, W_ref, b_ref)

    oh = oh_ref[...]
    dn = (((0,), (0,)), ((), ()))
    sum_scr[...] += lax.dot_general(oh, hout, dn,
                                    preferred_element_type=jnp.float32)
    cnt_scr[...] += lax.dot_general(
        oh, jnp.ones((RB, 128), jnp.float32), dn,
        preferred_element_type=jnp.float32)

    @pl.when(i == pl.num_programs(0) - 1)
    def _():
        pooled = sum_scr[...] / jnp.maximum(cnt_scr[:, :1], 1.0)
        out_ref[...] = (jnp.dot(pooled, lw_ref[...],
                                preferred_element_type=jnp.float32)
                        + lb_ref[...])


def _tc_last_layer(h, agg, cnt, root, W, b, onehotT, lin_w, lin_b):
    din = h.shape[1]
    ncls = lin_w.shape[1]
    return pl.pallas_call(
        _last_layer_body,
        grid=(NP // RB,),
        in_specs=[
            pl.BlockSpec((RB, din), lambda i: (i, 0)),
            pl.BlockSpec((RB, NUM_REL, din), lambda i: (i, 0, 0)),
            pl.BlockSpec((RB, NUM_REL, 128), lambda i: (i, 0, DIN // 128)),
            pl.BlockSpec((din, HIDDEN), lambda i: (0, 0)),
            pl.BlockSpec((NUM_REL, din, HIDDEN), lambda i: (0, 0, 0)),
            pl.BlockSpec((1, HIDDEN), lambda i: (0, 0)),
            pl.BlockSpec((RB, NUM_GRAPHS), lambda i: (i, 0)),
            pl.BlockSpec((HIDDEN, ncls), lambda i: (0, 0)),
            pl.BlockSpec((1, ncls), lambda i: (0, 0)),
        ],
        out_specs=pl.BlockSpec((NUM_GRAPHS, ncls), lambda i: (0, 0)),
        out_shape=jax.ShapeDtypeStruct((NUM_GRAPHS, ncls), jnp.float32),
        scratch_shapes=[
            pltpu.VMEM((NUM_GRAPHS, HIDDEN), jnp.float32),
            pltpu.VMEM((NUM_GRAPHS, 128), jnp.float32),
        ],
    )(h, agg, cnt, root, W, b, onehotT, lin_w, lin_b)


def kernel(x, edge_index, edge_attr, edge_type, batch, W1, root1, b1,
           W2, root2, b2, W3, root3, b3, lin_w, lin_b):
    src, dst = edge_index[0], edge_index[1]
    xp = jnp.zeros((NP, DIN), jnp.float32).at[:N_NODES].set(x)
    xa = (jnp.zeros((NP, DINA), jnp.float32).at[:N_NODES, :DIN].set(x)
          .at[:, DIN].set(1.0))
    batch_pad = jnp.full((NP,), NUM_GRAPHS, jnp.int32).at[:N_NODES].set(batch)
    onehotT = (batch_pad[:, None]
               == jnp.arange(NUM_GRAPHS)[None, :]).astype(jnp.float32)

    def _jnp_agg(h):
        g = h[src]
        return jnp.stack(
            [jax.ops.segment_sum(g * (edge_type == r).astype(jnp.float32)[:, None],
                                 dst, num_segments=NP) for r in range(NUM_REL)],
            axis=1)

    cnts = jnp.stack(
        [jax.ops.segment_sum((edge_type == r).astype(jnp.float32), dst,
                             num_segments=NP) for r in range(NUM_REL)],
        axis=1)
    cnt_arr = jnp.zeros((NP, NUM_REL, DINA), jnp.float32).at[:, :, DIN].set(cnts)

    agg1 = _jnp_agg(xp)
    h1 = _tc_mid_layer(xp, agg1, cnt_arr, root1, W1, b1.reshape(1, HIDDEN))
    agg2 = _jnp_agg(h1)
    h2 = _tc_mid_layer(h1, agg2, cnt_arr, root2, W2, b2.reshape(1, HIDDEN))
    agg3 = _jnp_agg(h2)
    return _tc_last_layer(h2, agg3, cnt_arr, root3, W3, b3.reshape(1, HIDDEN),
                          onehotT, lin_w, lin_b.reshape(1, -1))
